# baseline (speedup 1.0000x reference)
import jax
import jax.numpy as jnp
from jax import lax
from jax.experimental import pallas as pl
from jax.experimental.pallas import tpu as pltpu

_N = 64
_D = 128
_NEG = -1e30

_DN_STD = (((1,), (0,)), ((), ()))
_DN_LAST = (((1,), (1,)), ((), ()))


def _lane_sum(x):
    for sh in (64, 32, 16, 8, 4, 2, 1):
        x = x + pltpu.roll(x, sh, axis=1)
    return x


def _lane_min(x):
    for sh in (64, 32, 16, 8, 4, 2, 1):
        x = jnp.minimum(x, pltpu.roll(x, sh, axis=1))
    return x


def _gen_body(z_ref, w1_ref, w2_ref, b2_ref, wes_ref, gwcat_ref,
              sm4_ref, bias_ref, consts_ref, out_ref):
    z = z_ref[...]
    w1z = w1_ref[:, :_D]
    w1s = w1_ref[:, _D:]
    w2row = w2_ref[...]
    b2r = b2_ref[...]
    wez = wes_ref[0:1, :]
    wes = wes_ref[1:2, :]
    sm4 = sm4_ref[...]
    bias_mean = bias_ref[...]
    b1s = consts_ref[0, 0]
    bes = consts_ref[0, 1]
    w2we = consts_ref[0, 2]
    b2we = consts_ref[0, 3]

    row_i = lax.broadcasted_iota(jnp.int32, (_N, 1), 0)
    row_f = row_i.astype(jnp.float32)
    lane_i = lax.broadcasted_iota(jnp.int32, (1, _D), 1)
    lane64_f = lax.broadcasted_iota(jnp.int32, (1, _N), 1).astype(jnp.float32)

    zw1_row = _lane_sum(z * w1z) + b1s
    zwe_row = _lane_sum(z * wez) + bes

    def gat_big(hh):
        big = lax.dot_general(hh, gwcat_ref[...], _DN_STD,
                              preferred_element_type=jnp.float32)
        smr = lax.dot_general(sm4, hh, _DN_LAST,
                              preferred_element_type=jnp.float32)
        return big, smr

    def gat_rest(big, smr, krow, n2f):
        mask = row_f < krow[:, :_N]
        alphas = []
        for head in range(3):
            epre = big[:, 3 * _D + head:3 * _D + head + 1] \
                + smr[head:head + 1, :]
            e = jnp.where(epre >= 0, epre, 0.2 * epre)
            em = jnp.where(mask, e, _NEG)
            m = jnp.max(em, axis=1, keepdims=True)
            m = jnp.where(m > 0.1 * _NEG, m, 0.0)
            ex = jnp.where(mask, jnp.exp(e - m), 0.0)
            denom = jnp.sum(ex, axis=1, keepdims=True)
            dsafe = jnp.where(denom > 0, denom, 1.0)
            alphas.append(ex / dsafe)
        a_cat = jnp.concatenate(alphas, axis=1)
        f_cat = jnp.concatenate(
            [big[:, 0:_D], big[:, _D:2 * _D], big[:, 2 * _D:3 * _D]],
            axis=0)
        acc = lax.dot_general(a_cat, f_cat, _DN_STD,
                              preferred_element_type=jnp.float32)
        hnew = acc * (1.0 / 3.0) + bias_mean
        hnew = jnp.where(row_f < n2f, hnew, 0.0)
        snew = jnp.sum(hnew, axis=0, keepdims=True) / n2f
        return hnew, snew

    t0_row = jnp.maximum(zw1_row + _lane_sum(z * w1s), 0.0)
    h0row = t0_row * w2row + b2r
    h0 = jnp.where(row_i == 0, h0row, 0.0)
    k0 = jnp.zeros((1, _D), jnp.float32)
    big0, smr0 = gat_big(h0)
    h0, s0 = gat_rest(big0, smr0, k0, jnp.float32(1.0))

    def cond(c):
        return jnp.logical_not(c[4])

    def body(c):
        h, krow, n, s, _ = c
        tpre_row = zw1_row + _lane_sum(s * w1s)
        stop = jnp.logical_or(tpre_row[0, 0] <= 0.0, n >= _N)
        tok_row = jnp.maximum(tpre_row, 0.0)
        hnrow = tok_row * w2row + b2r
        h2 = jnp.where(row_i == n, hnrow, h)
        n2 = n + 1
        n2f = n2.astype(jnp.float32)
        cb_row = zwe_row + _lane_sum(s * wes) + tok_row * w2we + b2we
        big, smr = gat_big(h2)
        te_row = smr[3:4, :] + cb_row[:, :_N]
        brk = te_row < 1e-4
        cand = jnp.concatenate(
            [jnp.where(brk, lane64_f, jnp.float32(_N)),
             jnp.full((1, _N), jnp.float32(_N))], axis=1)
        knew = jnp.minimum(_lane_min(cand), n2f)
        krow2 = jnp.where(lane_i == n, knew, krow)
        hg, s3 = gat_rest(big, smr, krow2, n2f)
        h_o = jnp.where(stop, h, hg)
        k_o = jnp.where(stop, krow, krow2)
        n_o = jnp.where(stop, n, n2)
        s_o = jnp.where(stop, s, s3)
        return (h_o, k_o, n_o, s_o, stop)

    final = lax.while_loop(cond, body,
                           (h0, k0, jnp.int32(1), s0, jnp.bool_(False)))
    out_ref[...] = final[0]


def kernel(z, W1, b1, W2, b2, We, be, gat_W, gat_b, attn_l, attn_r):
    f32 = jnp.float32
    al3 = attn_l.reshape(3, _D).astype(f32)
    ar3 = attn_r.reshape(3, _D).astype(f32)
    gw3 = gat_W.astype(f32).reshape(3, _D, _D)
    galmT = jnp.einsum('hc,hck->hk', al3, gw3)
    garm = jnp.einsum('hc,hck->hk', ar3, gw3)
    we4_ = We.reshape(4, _D).astype(f32)
    gwcat = jnp.concatenate([
        gat_W.astype(f32),
        garm,
        we4_[3:4, :],
        jnp.zeros((512 - 384 - 4, _D), f32),
    ], axis=0).T
    sm4 = jnp.concatenate([galmT, we4_[3:4, :]], axis=0)
    gb3 = gat_b.reshape(3, _D).astype(f32)
    bias_mean = jnp.mean(gb3, axis=0, keepdims=True)
    w2row = W2.reshape(1, _D).astype(f32)
    b2r = b2.reshape(1, _D).astype(f32)
    consts = jnp.stack([
        b1.reshape(()).astype(f32),
        be.reshape(()).astype(f32),
        jnp.sum(w2row[0] * we4_[2]),
        jnp.sum(b2r[0] * we4_[2]),
    ]).reshape(1, 4)
    vmem = pl.BlockSpec(memory_space=pltpu.VMEM)
    smem = pl.BlockSpec(memory_space=pltpu.SMEM)
    return pl.pallas_call(
        _gen_body,
        out_shape=jax.ShapeDtypeStruct((_N, _D), f32),
        in_specs=[vmem] * 8 + [smem],
        out_specs=pl.BlockSpec(memory_space=pltpu.VMEM),
    )(
        z.astype(f32),
        W1.astype(f32),
        w2row,
        b2r,
        we4_[0:2, :],
        gwcat,
        sm4,
        bias_mean,
        consts,
    )


# speedup vs baseline: 2.0892x; 2.0892x over previous
import jax
import jax.numpy as jnp
from jax import lax
from jax.experimental import pallas as pl
from jax.experimental.pallas import tpu as pltpu

_N = 64
_D = 128
_NEG = -1e30

_DN_STD = (((1,), (0,)), ((), ()))
_DN_LAST = (((1,), (1,)), ((), ()))
_DN_S0 = (((0,), (0,)), ((), ()))


def _gen_body(z_ref, w1_ref, w2_ref, b2_ref, wes_ref, gwcat_ref,
              garm_ref, bias_ref, consts_ref, out_ref):
    z = z_ref[...]
    w1z = w1_ref[:, :_D]
    w1s = w1_ref[:, _D:]
    w2row = w2_ref[...]
    b2r = b2_ref[...]
    wez = wes_ref[0:1, :]
    wes = wes_ref[1:2, :]
    garm = garm_ref[...]
    bias_mean = bias_ref[...]
    b1s = consts_ref[0, 0]
    bes = consts_ref[0, 1]
    w2we = consts_ref[0, 2]
    b2we = consts_ref[0, 3]

    row_i = lax.broadcasted_iota(jnp.int32, (_N, 1), 0)
    row_f = row_i.astype(jnp.float32)
    d_row_f = lax.broadcasted_iota(jnp.int32, (1, _N), 1).astype(jnp.float32)

    zw1 = jnp.sum(z * w1z) + b1s
    zwe = jnp.sum(z * wez) + bes

    def gat_big(hh):
        big = lax.dot_general(hh, gwcat_ref[...], _DN_STD,
                              preferred_element_type=jnp.float32)
        er3 = lax.dot_general(garm, hh, _DN_LAST,
                              preferred_element_type=jnp.float32)
        return big, er3

    def gat_rest(big, er3, kcol, n2f):
        mask = d_row_f < kcol
        acc = jnp.zeros((_N, _D), jnp.float32)
        for head in range(3):
            epre = big[:, 3 * _D + head:3 * _D + head + 1] \
                + er3[head:head + 1, :]
            e = jnp.where(epre >= 0, epre, 0.2 * epre)
            em = jnp.where(mask, e, _NEG)
            m = jnp.max(em, axis=0, keepdims=True)
            m = jnp.where(m > 0.1 * _NEG, m, 0.0)
            ex = jnp.where(mask, jnp.exp(e - m), 0.0)
            denom = jnp.sum(ex, axis=0, keepdims=True)
            dsafe = jnp.where(denom > 0, denom, 1.0)
            alpha = ex / dsafe
            fh = big[:, head * _D:(head + 1) * _D]
            acc = acc + lax.dot_general(alpha, fh, _DN_S0,
                                        preferred_element_type=jnp.float32)
        hnew = acc * (1.0 / 3.0) + bias_mean
        hnew = jnp.where(row_f < n2f, hnew, 0.0)
        snew = jnp.sum(hnew, axis=0, keepdims=True) / n2f
        return hnew, snew

    t0 = jnp.maximum(zw1 + jnp.sum(z * w1s), 0.0)
    h0row = t0 * w2row + b2r
    h0 = jnp.where(row_i == 0, h0row, 0.0)
    k0 = jnp.zeros((_N, 1), jnp.float32)
    big0, er30 = gat_big(h0)
    h0, s0 = gat_rest(big0, er30, k0, jnp.float32(1.0))

    def cond(c):
        return jnp.logical_not(c[4])

    def body(c):
        h, kcol, n, s, _ = c
        tpre = zw1 + jnp.sum(s * w1s)
        stop = jnp.logical_or(tpre <= 0.0, n >= _N)
        tok = jnp.maximum(tpre, 0.0)
        hnrow = tok * w2row + b2r
        h2 = jnp.where(row_i == n, hnrow, h)
        n2 = n + 1
        n2f = n2.astype(jnp.float32)
        cbase = zwe + jnp.sum(s * wes) + tok * w2we + b2we
        big, er3 = gat_big(h2)
        te = big[:, 3 * _D + 3:3 * _D + 4] + cbase
        brk = te < 1e-4
        cand = jnp.where(brk, row_f, jnp.float32(_N))
        knew = jnp.minimum(jnp.min(cand), n2f)
        kcol2 = jnp.where(row_i == n, knew, kcol)
        hg, s3 = gat_rest(big, er3, kcol2, n2f)
        h_o = jnp.where(stop, h, hg)
        k_o = jnp.where(stop, kcol, kcol2)
        n_o = jnp.where(stop, n, n2)
        s_o = jnp.where(stop, s, s3)
        return (h_o, k_o, n_o, s_o, stop)

    final = lax.while_loop(cond, body,
                           (h0, k0, jnp.int32(1), s0, jnp.bool_(False)))
    out_ref[...] = final[0]


def kernel(z, W1, b1, W2, b2, We, be, gat_W, gat_b, attn_l, attn_r):
    f32 = jnp.float32
    al3 = attn_l.reshape(3, _D).astype(f32)
    ar3 = attn_r.reshape(3, _D).astype(f32)
    gw3 = gat_W.astype(f32).reshape(3, _D, _D)
    galmT = jnp.einsum('hc,hck->hk', al3, gw3)
    garm = jnp.einsum('hc,hck->hk', ar3, gw3)
    we4_ = We.reshape(4, _D).astype(f32)
    gwcat = jnp.concatenate([
        gat_W.astype(f32),
        galmT,
        we4_[3:4, :],
        jnp.zeros((512 - 384 - 4, _D), f32),
    ], axis=0).T
    gb3 = gat_b.reshape(3, _D).astype(f32)
    bias_mean = jnp.mean(gb3, axis=0, keepdims=True)
    w2row = W2.reshape(1, _D).astype(f32)
    b2r = b2.reshape(1, _D).astype(f32)
    consts = jnp.stack([
        b1.reshape(()).astype(f32),
        be.reshape(()).astype(f32),
        jnp.sum(w2row[0] * we4_[2]),
        jnp.sum(b2r[0] * we4_[2]),
    ]).reshape(1, 4)
    vmem = pl.BlockSpec(memory_space=pltpu.VMEM)
    smem = pl.BlockSpec(memory_space=pltpu.SMEM)
    return pl.pallas_call(
        _gen_body,
        out_shape=jax.ShapeDtypeStruct((_N, _D), f32),
        in_specs=[vmem] * 8 + [smem],
        out_specs=pl.BlockSpec(memory_space=pltpu.VMEM),
    )(
        z.astype(f32),
        W1.astype(f32),
        w2row,
        b2r,
        we4_[0:2, :],
        gwcat,
        garm,
        bias_mean,
        consts,
    )


# speedup vs baseline: 2.1992x; 1.0526x over previous
import jax
import jax.numpy as jnp
from jax import lax
from jax.experimental import pallas as pl
from jax.experimental.pallas import tpu as pltpu

_N = 64
_D = 128
_NEG = -1e30

_DN_STD = (((1,), (0,)), ((), ()))
_DN_LAST = (((1,), (1,)), ((), ()))
_DN_S0 = (((0,), (0,)), ((), ()))


def _gen_body(z_ref, w1_ref, wes_ref, gwcat_ref, garm_ref, bias_ref,
              dbig_ref, der_ref, consts_ref, out_ref):
    z = z_ref[...]
    w1z = w1_ref[:, :_D]
    w1s = w1_ref[:, _D:]
    wez = wes_ref[0:1, :]
    wes = wes_ref[1:2, :]
    garm = garm_ref[...]
    bias_mean = bias_ref[...]
    w2big = dbig_ref[0:1, :]
    b2big = dbig_ref[1:2, :]
    bias_big = dbig_ref[2:3, :]
    w2er = der_ref[:, 0:1]
    b2er = der_ref[:, 1:2]
    bias_er = der_ref[:, 2:3]
    b1s = consts_ref[0, 0]
    bes = consts_ref[0, 1]
    w2we = consts_ref[0, 2]
    b2we = consts_ref[0, 3]

    row_i = lax.broadcasted_iota(jnp.int32, (_N, 1), 0)
    row_f = row_i.astype(jnp.float32)
    lane_i = lax.broadcasted_iota(jnp.int32, (1, _N), 1)
    d_row_f = lane_i.astype(jnp.float32)

    zw1 = jnp.sum(z * w1z, axis=1, keepdims=True) + b1s
    zwe = jnp.sum(z * wez, axis=1, keepdims=True) + bes

    def proj(hh):
        big = lax.dot_general(hh, gwcat_ref[...], _DN_STD,
                              preferred_element_type=jnp.float32)
        er3 = lax.dot_general(garm, hh, _DN_LAST,
                              preferred_element_type=jnp.float32)
        return big, er3

    def gat_rest(big, er3, kcol, n2f):
        mask = d_row_f < kcol
        acc = jnp.zeros((_N, _D), jnp.float32)
        for head in range(3):
            epre = big[:, 3 * _D + head:3 * _D + head + 1] \
                + er3[head:head + 1, :]
            e = jnp.where(epre >= 0, epre, 0.2 * epre)
            em = jnp.where(mask, e, _NEG)
            m = jnp.max(em, axis=0, keepdims=True)
            m = jnp.where(m > 0.1 * _NEG, m, 0.0)
            ex = jnp.exp(em - m)
            denom = jnp.sum(ex, axis=0, keepdims=True)
            dsafe = jnp.where(denom > 0, denom, 1.0)
            alpha = ex / dsafe
            fh = big[:, head * _D:(head + 1) * _D]
            acc = acc + lax.dot_general(alpha, fh, _DN_S0,
                                        preferred_element_type=jnp.float32)
        hnew = acc * (1.0 / 3.0) + bias_mean
        hnew = jnp.where(row_f < n2f, hnew, 0.0)
        snew = jnp.sum(hnew, axis=0, keepdims=True) / n2f
        return hnew, snew

    h0 = jnp.where(row_i == 0, bias_mean, 0.0)
    s0 = bias_mean
    big0 = jnp.where(row_i == 0, bias_big, 0.0)
    er30 = jnp.where(lane_i == 0, bias_er, 0.0)
    k0 = jnp.zeros((_N, 1), jnp.float32)

    def cond(c):
        return jnp.logical_not(c[6])

    def body(c):
        h, big, er3, kcol, n, s, _ = c
        tpre = zw1 + jnp.sum(s * w1s, axis=1, keepdims=True)
        stop = jnp.logical_or(tpre[0, 0] <= 0.0, n >= _N)
        tok = jnp.maximum(tpre, 0.0)
        big2 = big + jnp.where(row_i == n, tok * w2big + b2big, 0.0)
        er32 = er3 + jnp.where(lane_i == n, tok * w2er + b2er, 0.0)
        n2 = n + 1
        n2f = n2.astype(jnp.float32)
        cbase = zwe + jnp.sum(s * wes, axis=1, keepdims=True) \
            + tok * w2we + b2we
        te = big2[:, 3 * _D + 3:3 * _D + 4] + cbase
        brk = te < 1e-4
        cand = jnp.where(brk, row_f, jnp.float32(_N))
        knew = jnp.minimum(jnp.min(cand, axis=0, keepdims=True),
                           n2f)
        kcol2 = jnp.where(row_i == n, knew, kcol)
        hg, s3 = gat_rest(big2, er32, kcol2, n2f)
        bg, erg = proj(hg)
        h_o = jnp.where(stop, h, hg)
        big_o = jnp.where(stop, big, bg)
        er_o = jnp.where(stop, er3, erg)
        k_o = jnp.where(stop, kcol, kcol2)
        n_o = jnp.where(stop, n, n2)
        s_o = jnp.where(stop, s, s3)
        return (h_o, big_o, er_o, k_o, n_o, s_o, stop)

    final = lax.while_loop(
        cond, body,
        (h0, big0, er30, k0, jnp.int32(1), s0, jnp.bool_(False)))
    out_ref[...] = final[0]


def kernel(z, W1, b1, W2, b2, We, be, gat_W, gat_b, attn_l, attn_r):
    f32 = jnp.float32
    al3 = attn_l.reshape(3, _D).astype(f32)
    ar3 = attn_r.reshape(3, _D).astype(f32)
    gw3 = gat_W.astype(f32).reshape(3, _D, _D)
    galmT = jnp.einsum('hc,hck->hk', al3, gw3)
    garm = jnp.einsum('hc,hck->hk', ar3, gw3)
    we4_ = We.reshape(4, _D).astype(f32)
    gwcat = jnp.concatenate([
        gat_W.astype(f32),
        galmT,
        we4_[3:4, :],
        jnp.zeros((512 - 384 - 4, _D), f32),
    ], axis=0).T
    gb3 = gat_b.reshape(3, _D).astype(f32)
    bias_mean = jnp.mean(gb3, axis=0, keepdims=True)
    w2row = W2.reshape(1, _D).astype(f32)
    b2r = b2.reshape(1, _D).astype(f32)
    gens = jnp.concatenate([w2row, b2r, bias_mean], axis=0)
    dbig = gens @ gwcat
    der = lax.dot_general(garm, gens, _DN_LAST)
    consts = jnp.stack([
        b1.reshape(()).astype(f32),
        be.reshape(()).astype(f32),
        jnp.sum(w2row[0] * we4_[2]),
        jnp.sum(b2r[0] * we4_[2]),
    ]).reshape(1, 4)
    vmem = pl.BlockSpec(memory_space=pltpu.VMEM)
    smem = pl.BlockSpec(memory_space=pltpu.SMEM)
    return pl.pallas_call(
        _gen_body,
        out_shape=jax.ShapeDtypeStruct((_N, _D), f32),
        in_specs=[vmem] * 8 + [smem],
        out_specs=pl.BlockSpec(memory_space=pltpu.VMEM),
    )(
        z.astype(f32),
        W1.astype(f32),
        we4_[0:2, :],
        gwcat,
        garm,
        bias_mean,
        dbig,
        der,
        consts,
    )
